# baseline (speedup 1.0000x reference)
import functools

import jax
import jax.numpy as jnp
from jax import lax
from jax.experimental import pallas as pl
from jax.experimental.pallas import tpu as pltpu
from jax.experimental.pallas import tpu_sc as plsc

_NC = 2
_NS = 16
_NW = _NC * _NS
_LANES = 16


def _emb_body(L, D, rows_per_worker,
              ids_hbm, table_hbm, pos_hbm, out_hbm,
              pos_v, idx_v, rows_v, sem):
    wid = lax.axis_index("s") * _NC + lax.axis_index("c")
    pltpu.sync_copy(pos_hbm, pos_v)

    def row_body(i, carry):
        b = wid * rows_per_worker + i
        pltpu.sync_copy(ids_hbm.at[b], idx_v)
        pltpu.async_copy(table_hbm.at[idx_v], rows_v, sem).wait()

        def add_body(l, c):
            for k in range(D // _LANES):
                sl = pl.ds(k * _LANES, _LANES)
                rows_v[l, sl] = rows_v[l, sl] + pos_v[l, sl]
            return c

        lax.fori_loop(0, L, add_body, 0, unroll=2)
        pltpu.sync_copy(rows_v, out_hbm.at[b])
        return carry

    lax.fori_loop(0, rows_per_worker, row_body, 0)


def kernel(input_ids, content_table, pos_table):
    B, L = input_ids.shape
    V, D = content_table.shape
    assert B % _NW == 0 and D % _LANES == 0
    rows_per_worker = B // _NW
    ids = input_ids.astype(jnp.int32)

    mesh = plsc.VectorSubcoreMesh(core_axis_name="c", subcore_axis_name="s")
    k = functools.partial(
        pl.kernel,
        out_type=jax.ShapeDtypeStruct((B, L, D), jnp.float32),
        mesh=mesh,
        compiler_params=pltpu.CompilerParams(use_tc_tiling_on_sc=False),
        scratch_types=[
            pltpu.VMEM((L, D), jnp.float32),
            pltpu.VMEM((L,), jnp.int32),
            pltpu.VMEM((L, D), jnp.float32),
            pltpu.SemaphoreType.DMA,
        ],
    )(functools.partial(_emb_body, L, D, rows_per_worker))
    return k(ids, content_table, pos_table)


# speedup vs baseline: 1.1423x; 1.1423x over previous
import functools

import jax
import jax.numpy as jnp
from jax import lax
from jax.experimental import pallas as pl
from jax.experimental.pallas import tpu as pltpu
from jax.experimental.pallas import tpu_sc as plsc

_NC = 2
_NS = 16
_NW = _NC * _NS
_LANES = 16


def _emb_body(L, D, rpw,
              ids_hbm, table_hbm, pos_hbm, out_hbm,
              pos_v, ids_v,
              gbuf0, gbuf1, sbuf0, sbuf1,
              gsem0, gsem1, ssem0, ssem1):
    wid = lax.axis_index("s") * _NC + lax.axis_index("c")
    base = wid * rpw
    pltpu.sync_copy(pos_hbm, pos_v)
    pltpu.sync_copy(ids_hbm.at[pl.ds(base * L, rpw * L)], ids_v)

    gbufs = (gbuf0, gbuf1)
    sbufs = (sbuf0, sbuf1)
    gsems = (gsem0, gsem1)
    ssems = (ssem0, ssem1)

    def gather_desc(c, s):
        idx = ids_v.at[pl.ds(c * L, L)]
        return pltpu.make_async_copy(table_hbm.at[idx], gbufs[s], gsems[s])

    def scatter_desc(c, s):
        return pltpu.make_async_copy(sbufs[s], out_hbm.at[base + c], ssems[s])

    gather_desc(0, 0).start()
    gather_desc(1, 1).start()

    def pair_body(c2, carry):
        for s in range(2):
            c = 2 * c2 + s
            gather_desc(c, s).wait()

            @pl.when(c2 >= 1)
            def _wait_prev_scatter():
                scatter_desc(c - 2, s).wait()

            gbuf, sbuf = gbufs[s], sbufs[s]

            def add_body(l, acc):
                for k in range(D // _LANES):
                    sl = pl.ds(k * _LANES, _LANES)
                    sbuf[l, sl] = gbuf[l, sl] + pos_v[l, sl]
                return acc

            lax.fori_loop(0, L, add_body, 0, unroll=4)
            scatter_desc(c, s).start()

            @pl.when(c2 < rpw // 2 - 1)
            def _prefetch_gather():
                gather_desc(c + 2, s).start()
        return carry

    lax.fori_loop(0, rpw // 2, pair_body, 0)
    scatter_desc(rpw - 2, 0).wait()
    scatter_desc(rpw - 1, 1).wait()


def kernel(input_ids, content_table, pos_table):
    B, L = input_ids.shape
    V, D = content_table.shape
    assert B % (2 * _NW) == 0 and D % _LANES == 0
    rpw = B // _NW
    ids = input_ids.astype(jnp.int32).reshape(B * L)

    mesh = plsc.VectorSubcoreMesh(core_axis_name="c", subcore_axis_name="s")
    k = functools.partial(
        pl.kernel,
        out_type=jax.ShapeDtypeStruct((B, L, D), jnp.float32),
        mesh=mesh,
        compiler_params=pltpu.CompilerParams(use_tc_tiling_on_sc=False),
        scratch_types=[
            pltpu.VMEM((L, D), jnp.float32),
            pltpu.VMEM((rpw * L,), jnp.int32),
            pltpu.VMEM((L, D), jnp.float32),
            pltpu.VMEM((L, D), jnp.float32),
            pltpu.VMEM((L, D), jnp.float32),
            pltpu.VMEM((L, D), jnp.float32),
            pltpu.SemaphoreType.DMA,
            pltpu.SemaphoreType.DMA,
            pltpu.SemaphoreType.DMA,
            pltpu.SemaphoreType.DMA,
        ],
    )(functools.partial(_emb_body, L, D, rpw))
    return k(ids, content_table, pos_table)


# speedup vs baseline: 1.3547x; 1.1859x over previous
import functools

import jax
import jax.numpy as jnp
from jax import lax
from jax.experimental import pallas as pl
from jax.experimental.pallas import tpu as pltpu
from jax.experimental.pallas import tpu_sc as plsc

_NC = 2
_NS = 16
_NW = _NC * _NS
_CH = 512


def _gather_body(N, D, ids_hbm, tab_hbm, out_hbm,
                 ids_v, buf0, buf1, gsem0, gsem1, ssem0, ssem1):
    wid = lax.axis_index("s") * _NC + lax.axis_index("c")
    per_w = N // _NW
    nch = per_w // _CH
    base = wid * per_w
    bufs = (buf0, buf1)
    gsems = (gsem0, gsem1)
    ssems = (ssem0, ssem1)

    pltpu.sync_copy(ids_hbm.at[pl.ds(base, per_w)], ids_v)

    def gather_desc(c, slot):
        idx = ids_v.at[pl.ds(c * _CH, _CH)]
        return pltpu.make_async_copy(
            tab_hbm.at[idx], bufs[slot], gsems[slot])

    def scatter_desc(c, slot):
        return pltpu.make_async_copy(
            bufs[slot], out_hbm.at[pl.ds(base + c * _CH, _CH), :],
            ssems[slot])

    gather_desc(0, 0).start()
    gather_desc(1, 1).start()

    def chunk_pair(c2, carry):
        for slot in range(2):
            c = 2 * c2 + slot
            gather_desc(c, slot).wait()
            scatter_desc(c, slot).start()

            @pl.when(c2 < nch // 2 - 1)
            def _():
                scatter_desc(c, slot).wait()
                gather_desc(c + 2, slot).start()
        return carry

    lax.fori_loop(0, nch // 2, chunk_pair, 0)
    scatter_desc(nch - 2, 0).wait()
    scatter_desc(nch - 1, 1).wait()


def kernel(input_ids, content_table, pos_table):
    B, L = input_ids.shape
    V, D = content_table.shape
    N = B * L
    assert N % (_NW * _CH) == 0
    ids = input_ids.astype(jnp.int32).reshape(N)

    mesh = plsc.VectorSubcoreMesh(core_axis_name="c", subcore_axis_name="s")
    k = functools.partial(
        pl.kernel,
        out_type=jax.ShapeDtypeStruct((N, D), jnp.float32),
        mesh=mesh,
        compiler_params=pltpu.CompilerParams(use_tc_tiling_on_sc=False),
        scratch_types=[
            pltpu.VMEM((N // _NW,), jnp.int32),
            pltpu.VMEM((_CH, D), jnp.float32),
            pltpu.VMEM((_CH, D), jnp.float32),
            pltpu.SemaphoreType.DMA,
            pltpu.SemaphoreType.DMA,
            pltpu.SemaphoreType.DMA,
            pltpu.SemaphoreType.DMA,
        ],
    )(functools.partial(_gather_body, N, D))
    content = k(ids, content_table)
    return content.reshape(B, L, D) + pos_table[None, :, :]
